# baseline (speedup 1.0000x reference)
import functools

import jax
import jax.numpy as jnp
from jax import lax
from jax.experimental import pallas as pl
from jax.experimental.pallas import tpu as pltpu
from jax.experimental.pallas import tpu_sc as plsc

N = 320000
C = 128
G = 10000
NC = 2
NS = 16
NW = NC * NS
RPW = N // NW
CH = 80
NCH = RPW // CH
NSEG_CH = G // CH
SEG_ROUNDS = -(-NSEG_CH // NS)
SEG_ROUNDS_W = -(-NSEG_CH // NW)
L = 16

_mesh = plsc.VectorSubcoreMesh(
    core_axis_name="c", subcore_axis_name="s", num_cores=NC, num_subcores=NS)


def _fill_const(buf, val):
    def _fill(i, _):
        for v in range(C // L):
            buf[i, pl.ds(v * L, L)] = jnp.full((L,), val, jnp.float32)
        return _
    lax.fori_loop(0, CH, _fill, None)


NB = 3
NBS = 3


def _make_scatter_kernel(ones_mode):
    scratch = [
        pltpu.VMEM_SHARED((G, C), jnp.float32),
        pltpu.VMEM((NCH, CH), jnp.int32),
        pltpu.VMEM((CH, C), jnp.float32),
    ]
    if ones_mode:
        scratch += [pltpu.SemaphoreType.DMA]
    else:
        scratch += [pltpu.VMEM((CH, C), jnp.float32)
                    for _ in range(NBS - 1)]
        scratch += [pltpu.SemaphoreType.DMA for _ in range(NBS)]

    @functools.partial(
        pl.kernel,
        out_type=jax.ShapeDtypeStruct((NC, G, C), jnp.float32),
        mesh=_mesh,
        scratch_types=scratch,
    )
    def _scatter_kernel(feat_hbm, ids_hbm, acc_hbm, acc_sh, ids_v, fb, *ring):
        c = lax.axis_index("c")
        s = lax.axis_index("s")
        w = s * NC + c

        _fill_const(fb, 0.0)

        def _zero(k, _):
            ci = s + k * NS

            @pl.when(ci < NSEG_CH)
            def _():
                pltpu.sync_copy(fb, acc_sh.at[pl.ds(ci * CH, CH)])
            return _
        lax.fori_loop(0, SEG_ROUNDS, _zero, None)
        plsc.subcore_barrier()

        pltpu.sync_copy(ids_hbm.at[w], ids_v)

        if ones_mode:
            _fill_const(fb, 1.0)
            sem = ring[0]

            def _issue(j, _):
                pltpu.async_copy(fb, acc_sh.at[ids_v.at[j]], sem, add=True)
                return _
            lax.fori_loop(0, NCH, _issue, None)

            def _drain(j, _):
                pltpu.make_async_copy(
                    feat_hbm.at[pl.ds(0, CH)], fb, sem).wait()
                return _
            lax.fori_loop(0, NCH, _drain, None)
        else:
            bufs = (fb,) + ring[:NBS - 1]
            sems = ring[NBS - 1:]
            base = w * RPW
            nfull = NCH // NBS
            ntail = NCH % NBS
            for b in range(NBS):
                pltpu.async_copy(
                    feat_hbm.at[pl.ds(base + b * CH, CH)], bufs[b], sems[b])

            def _accum(k, _):
                for b in range(NBS):
                    j = k * NBS + b
                    pltpu.make_async_copy(
                        feat_hbm.at[pl.ds(0, CH)], bufs[b], sems[b]).wait()
                    pltpu.sync_copy(bufs[b], acc_sh.at[ids_v.at[j]], add=True)

                    @pl.when(j + NBS < NCH)
                    def _():
                        pltpu.async_copy(
                            feat_hbm.at[pl.ds(base + (j + NBS) * CH, CH)],
                            bufs[b], sems[b])
                return _
            lax.fori_loop(0, nfull, _accum, None)

            for t in range(ntail):
                pltpu.make_async_copy(
                    feat_hbm.at[pl.ds(0, CH)], bufs[t], sems[t]).wait()
                pltpu.sync_copy(
                    bufs[t], acc_sh.at[ids_v.at[NCH - ntail + t]], add=True)
        plsc.subcore_barrier()

        def _dump(k, _):
            ci = s + k * NS

            @pl.when(ci < NSEG_CH)
            def _():
                pltpu.sync_copy(acc_sh.at[pl.ds(ci * CH, CH)],
                                acc_hbm.at[c, pl.ds(ci * CH, CH)])
            return _
        lax.fori_loop(0, SEG_ROUNDS, _dump, None)

    return _scatter_kernel


_sum_partials_kernel = _make_scatter_kernel(ones_mode=False)
_count_partials_kernel = _make_scatter_kernel(ones_mode=True)


@functools.partial(
    pl.kernel,
    out_type=jax.ShapeDtypeStruct((G, C), jnp.float32),
    mesh=_mesh,
    scratch_types=[
        pltpu.VMEM((CH, C), jnp.float32),
        pltpu.VMEM((CH, C), jnp.float32),
        pltpu.VMEM((CH, C), jnp.float32),
        pltpu.VMEM((CH, C), jnp.float32),
        pltpu.VMEM((CH, C), jnp.float32),
    ],
)
def _means_kernel(ps_hbm, pc_hbm, mean_hbm, b0, b1, c0, c1, mb):
    c = lax.axis_index("c")
    s = lax.axis_index("s")
    w = s * NC + c

    def _combine(k, _):
        ci = w + k * NW

        @pl.when(ci < NSEG_CH)
        def _():
            segbase = ci * CH
            pltpu.sync_copy(ps_hbm.at[0, pl.ds(segbase, CH)], b0)
            pltpu.sync_copy(ps_hbm.at[1, pl.ds(segbase, CH)], b1)
            pltpu.sync_copy(pc_hbm.at[0, pl.ds(segbase, CH)], c0)
            pltpu.sync_copy(pc_hbm.at[1, pl.ds(segbase, CH)], c1)

            def _row(r, _2):
                for v in range(C // L):
                    sl = pl.ds(v * L, L)
                    cnt = jnp.maximum(c0[r, sl] + c1[r, sl], 1.0)
                    mb[r, sl] = (b0[r, sl] + b1[r, sl]) / cnt
                return _2
            lax.fori_loop(0, CH, _row, None)
            pltpu.sync_copy(mb, mean_hbm.at[pl.ds(segbase, CH)])
        return _
    lax.fori_loop(0, SEG_ROUNDS_W, _combine, None)


@functools.partial(
    pl.kernel,
    out_type=jax.ShapeDtypeStruct((N, C), jnp.float32),
    mesh=_mesh,
    scratch_types=[
        pltpu.VMEM_SHARED((G, C), jnp.float32),
        pltpu.VMEM((NCH, CH), jnp.int32),
    ] + [pltpu.VMEM((CH, C), jnp.float32) for _ in range(NB)]
      + [pltpu.SemaphoreType.DMA for _ in range(NB)],
)
def _gather_kernel(mean_hbm, ids_hbm, out_hbm, means_sh, ids_v, *ring):
    c = lax.axis_index("c")
    s = lax.axis_index("s")
    w = s * NC + c

    def _stage(k, _):
        ci = s + k * NS

        @pl.when(ci < NSEG_CH)
        def _():
            pltpu.sync_copy(mean_hbm.at[pl.ds(ci * CH, CH)],
                            means_sh.at[pl.ds(ci * CH, CH)])
        return _
    lax.fori_loop(0, SEG_ROUNDS, _stage, None)
    plsc.subcore_barrier()

    pltpu.sync_copy(ids_hbm.at[w], ids_v)
    bufs, sems = ring[:NB], ring[NB:]
    base = w * RPW
    nfull = NCH // NB
    ntail = NCH % NB

    for b in range(NB):
        pltpu.sync_copy(means_sh.at[ids_v.at[b]], bufs[b])
        pltpu.async_copy(bufs[b], out_hbm.at[pl.ds(base + b * CH, CH)],
                         sems[b])

    def _gather(k, _):
        for b in range(NB):
            j = k * NB + b
            pltpu.make_async_copy(
                mean_hbm.at[pl.ds(0, CH)], bufs[b], sems[b]).wait()
            pltpu.sync_copy(means_sh.at[ids_v.at[j]], bufs[b])
            pltpu.async_copy(bufs[b], out_hbm.at[pl.ds(base + j * CH, CH)],
                             sems[b])
        return _
    lax.fori_loop(1, nfull, _gather, None)

    for t in range(ntail):
        j = nfull * NB + t
        pltpu.make_async_copy(
            mean_hbm.at[pl.ds(0, CH)], bufs[t], sems[t]).wait()
        pltpu.sync_copy(means_sh.at[ids_v.at[j]], bufs[t])
        pltpu.sync_copy(bufs[t], out_hbm.at[pl.ds(base + j * CH, CH)])
    for t in range(ntail, NB):
        pltpu.make_async_copy(
            mean_hbm.at[pl.ds(0, CH)], bufs[t], sems[t]).wait()


def kernel(ref_bxyz, ref_feat, group_ids):
    del ref_bxyz
    ids3 = group_ids.reshape(NW, NCH, CH)
    ps = _sum_partials_kernel(ref_feat, ids3)
    pc = _count_partials_kernel(ref_feat, ids3)
    means = _means_kernel(ps, pc)
    return _gather_kernel(means, ids3)


# speedup vs baseline: 1.0270x; 1.0270x over previous
import functools

import jax
import jax.numpy as jnp
from jax import lax
from jax.experimental import pallas as pl
from jax.experimental.pallas import tpu as pltpu
from jax.experimental.pallas import tpu_sc as plsc

N = 320000
C = 128
G = 10000
NC = 2
NS = 16
NW = NC * NS
RPW = N // NW
CH = 80
NCH = RPW // CH
NSEG_CH = G // CH
SEG_ROUNDS = -(-NSEG_CH // NS)
SEG_ROUNDS_W = -(-NSEG_CH // NW)
L = 16

_mesh = plsc.VectorSubcoreMesh(
    core_axis_name="c", subcore_axis_name="s", num_cores=NC, num_subcores=NS)


def _fill_const(buf, val):
    def _fill(i, _):
        for v in range(C // L):
            buf[i, pl.ds(v * L, L)] = jnp.full((L,), val, jnp.float32)
        return _
    lax.fori_loop(0, CH, _fill, None)


NB = 3
NBS = 3


def _make_scatter_kernel(ones_mode):
    scratch = [
        pltpu.VMEM_SHARED((G, C), jnp.float32),
        pltpu.VMEM((NCH, CH), jnp.int32),
        pltpu.VMEM((CH, C), jnp.float32),
    ]
    if ones_mode:
        scratch += [pltpu.SemaphoreType.DMA]
    else:
        scratch += [pltpu.VMEM((CH, C), jnp.float32)
                    for _ in range(NBS - 1)]
        scratch += [pltpu.SemaphoreType.DMA for _ in range(NBS)]

    @functools.partial(
        pl.kernel,
        out_type=jax.ShapeDtypeStruct((NC, G, C), jnp.float32),
        mesh=_mesh,
        scratch_types=scratch,
    )
    def _scatter_kernel(feat_hbm, ids_hbm, acc_hbm, acc_sh, ids_v, fb, *ring):
        c = lax.axis_index("c")
        s = lax.axis_index("s")
        w = s * NC + c

        _fill_const(fb, 0.0)

        def _zero(k, _):
            ci = s + k * NS

            @pl.when(ci < NSEG_CH)
            def _():
                pltpu.sync_copy(fb, acc_sh.at[pl.ds(ci * CH, CH)])
            return _
        lax.fori_loop(0, SEG_ROUNDS, _zero, None)
        plsc.subcore_barrier()

        pltpu.sync_copy(ids_hbm.at[w], ids_v)

        if ones_mode:
            _fill_const(fb, 1.0)
            sem = ring[0]

            def _issue(j, _):
                pltpu.async_copy(fb, acc_sh.at[ids_v.at[j]], sem, add=True)
                return _
            lax.fori_loop(0, NCH, _issue, None)

            def _drain(j, _):
                pltpu.make_async_copy(
                    feat_hbm.at[pl.ds(0, CH)], fb, sem).wait()
                return _
            lax.fori_loop(0, NCH, _drain, None)
        else:
            bufs = (fb,) + ring[:NBS - 1]
            sems = ring[NBS - 1:]
            base = w * RPW
            nfull = NCH // NBS
            ntail = NCH % NBS
            for b in range(NBS):
                pltpu.async_copy(
                    feat_hbm.at[pl.ds(base + b * CH, CH)], bufs[b], sems[b])

            def _accum(k, _):
                for b in range(NBS):
                    j = k * NBS + b
                    pltpu.make_async_copy(
                        feat_hbm.at[pl.ds(0, CH)], bufs[b], sems[b]).wait()
                    pltpu.sync_copy(bufs[b], acc_sh.at[ids_v.at[j]], add=True)

                    @pl.when(j + NBS < NCH)
                    def _():
                        pltpu.async_copy(
                            feat_hbm.at[pl.ds(base + (j + NBS) * CH, CH)],
                            bufs[b], sems[b])
                return _
            lax.fori_loop(0, nfull, _accum, None)

            for t in range(ntail):
                pltpu.make_async_copy(
                    feat_hbm.at[pl.ds(0, CH)], bufs[t], sems[t]).wait()
                pltpu.sync_copy(
                    bufs[t], acc_sh.at[ids_v.at[NCH - ntail + t]], add=True)
        plsc.subcore_barrier()

        def _dump(k, _):
            ci = s + k * NS

            @pl.when(ci < NSEG_CH)
            def _():
                pltpu.sync_copy(acc_sh.at[pl.ds(ci * CH, CH)],
                                acc_hbm.at[c, pl.ds(ci * CH, CH)])
            return _
        lax.fori_loop(0, SEG_ROUNDS, _dump, None)

    return _scatter_kernel


_sum_partials_kernel = _make_scatter_kernel(ones_mode=False)
_count_partials_kernel = _make_scatter_kernel(ones_mode=True)


@functools.partial(
    pl.kernel,
    out_type=jax.ShapeDtypeStruct((G, C), jnp.float32),
    mesh=_mesh,
    scratch_types=[
        pltpu.VMEM((CH, C), jnp.float32),
        pltpu.VMEM((CH, C), jnp.float32),
        pltpu.VMEM((CH, C), jnp.float32),
        pltpu.VMEM((CH, C), jnp.float32),
        pltpu.VMEM((CH, C), jnp.float32),
        pltpu.SemaphoreType.DMA,
    ],
)
def _means_kernel(ps_hbm, pc_hbm, mean_hbm, b0, b1, c0, c1, mb, sem):
    c = lax.axis_index("c")
    s = lax.axis_index("s")
    w = s * NC + c

    def _combine(k, _):
        ci = w + k * NW

        @pl.when(ci < NSEG_CH)
        def _():
            segbase = ci * CH
            pltpu.async_copy(ps_hbm.at[0, pl.ds(segbase, CH)], b0, sem)
            pltpu.async_copy(ps_hbm.at[1, pl.ds(segbase, CH)], b1, sem)
            pltpu.async_copy(pc_hbm.at[0, pl.ds(segbase, CH)], c0, sem)
            pltpu.async_copy(pc_hbm.at[1, pl.ds(segbase, CH)], c1, sem)
            for buf in (b0, b1, c0, c1):
                pltpu.make_async_copy(
                    ps_hbm.at[0, pl.ds(0, CH)], buf, sem).wait()

            def _row(r, _2):
                for v in range(C // L):
                    sl = pl.ds(v * L, L)
                    cnt = jnp.maximum(c0[r, sl] + c1[r, sl], 1.0)
                    mb[r, sl] = (b0[r, sl] + b1[r, sl]) / cnt
                return _2
            lax.fori_loop(0, CH, _row, None)
            pltpu.sync_copy(mb, mean_hbm.at[pl.ds(segbase, CH)])
        return _
    lax.fori_loop(0, SEG_ROUNDS_W, _combine, None)


@functools.partial(
    pl.kernel,
    out_type=jax.ShapeDtypeStruct((N, C), jnp.float32),
    mesh=_mesh,
    scratch_types=[
        pltpu.VMEM_SHARED((G, C), jnp.float32),
        pltpu.VMEM((NCH, CH), jnp.int32),
    ] + [pltpu.VMEM((CH, C), jnp.float32) for _ in range(NB)]
      + [pltpu.SemaphoreType.DMA for _ in range(NB)],
)
def _gather_kernel(mean_hbm, ids_hbm, out_hbm, means_sh, ids_v, *ring):
    c = lax.axis_index("c")
    s = lax.axis_index("s")
    w = s * NC + c

    def _stage(k, _):
        ci = s + k * NS

        @pl.when(ci < NSEG_CH)
        def _():
            pltpu.sync_copy(mean_hbm.at[pl.ds(ci * CH, CH)],
                            means_sh.at[pl.ds(ci * CH, CH)])
        return _
    lax.fori_loop(0, SEG_ROUNDS, _stage, None)
    plsc.subcore_barrier()

    pltpu.sync_copy(ids_hbm.at[w], ids_v)
    bufs, sems = ring[:NB], ring[NB:]
    base = w * RPW
    nfull = NCH // NB
    ntail = NCH % NB

    for b in range(NB):
        pltpu.sync_copy(means_sh.at[ids_v.at[b]], bufs[b])
        pltpu.async_copy(bufs[b], out_hbm.at[pl.ds(base + b * CH, CH)],
                         sems[b])

    def _gather(k, _):
        for b in range(NB):
            j = k * NB + b
            pltpu.make_async_copy(
                mean_hbm.at[pl.ds(0, CH)], bufs[b], sems[b]).wait()
            pltpu.sync_copy(means_sh.at[ids_v.at[j]], bufs[b])
            pltpu.async_copy(bufs[b], out_hbm.at[pl.ds(base + j * CH, CH)],
                             sems[b])
        return _
    lax.fori_loop(1, nfull, _gather, None)

    for t in range(ntail):
        j = nfull * NB + t
        pltpu.make_async_copy(
            mean_hbm.at[pl.ds(0, CH)], bufs[t], sems[t]).wait()
        pltpu.sync_copy(means_sh.at[ids_v.at[j]], bufs[t])
        pltpu.sync_copy(bufs[t], out_hbm.at[pl.ds(base + j * CH, CH)])
    for t in range(ntail, NB):
        pltpu.make_async_copy(
            mean_hbm.at[pl.ds(0, CH)], bufs[t], sems[t]).wait()


def kernel(ref_bxyz, ref_feat, group_ids):
    del ref_bxyz
    ids3 = group_ids.reshape(NW, NCH, CH)
    ps = _sum_partials_kernel(ref_feat, ids3)
    pc = _count_partials_kernel(ref_feat, ids3)
    means = _means_kernel(ps, pc)
    return _gather_kernel(means, ids3)
